# baseline (speedup 1.0000x reference)
import functools

import jax
import jax.numpy as jnp
from jax import lax
from jax.experimental import pallas as pl
from jax.experimental.pallas import tpu as pltpu
from jax.experimental.pallas import tpu_sc as plsc

E = 8
TOPK = 2
M = 2048
K = 1024
FF = 1024
N = 2 * FF

B = 128
NB = (M * TOPK) // B + E
NBB = NB * B

NC, NS = 2, 16
NW = NC * NS

CT = 16
TPW = M // NW

_MESH = dict(core_axis_name="c", subcore_axis_name="s")


def _sc_scatter(hs, p0, p1):
  nchunks = TPW // CT

  @functools.partial(
      pl.kernel,
      out_type=jax.ShapeDtypeStruct((NBB, K), jnp.float32),
      mesh=plsc.VectorSubcoreMesh(**_MESH),
      scratch_types=[
          pltpu.VMEM((CT,), jnp.int32),
          pltpu.VMEM((CT,), jnp.int32),
          pltpu.VMEM((CT,), jnp.int32),
          pltpu.VMEM((CT,), jnp.int32),
          pltpu.VMEM((CT, K), jnp.float32),
          pltpu.VMEM((CT, K), jnp.float32),
          pltpu.SemaphoreType.DMA,
          pltpu.SemaphoreType.DMA,
          pltpu.SemaphoreType.DMA,
          pltpu.SemaphoreType.DMA,
      ],
  )
  def scatter_kernel(hs_hbm, p0_hbm, p1_hbm, out_hbm, i0a, i0b, i1a, i1b,
                     bufa, bufb, sla, slb, ssa, ssb):
    wid = lax.axis_index("s") * NC + lax.axis_index("c")
    base = wid * TPW
    bufs = (bufa, bufb)
    i0s = (i0a, i0b)
    i1s = (i1a, i1b)
    lsems = (sla, slb)
    ssems = (ssa, ssb)
    loads = [None] * nchunks
    scats = [None] * nchunks
    pltpu.sync_copy(p0_hbm.at[pl.ds(base, CT)], i0s[0])
    pltpu.sync_copy(p1_hbm.at[pl.ds(base, CT)], i1s[0])
    loads[0] = pltpu.async_copy(hs_hbm.at[pl.ds(base, CT)], bufs[0], lsems[0])
    for c in range(nchunks):
      pb = c % 2
      if c + 1 < nchunks:
        if c >= 1:
          scats[c - 1][0].wait()
          scats[c - 1][1].wait()
        off = base + (c + 1) * CT
        pltpu.sync_copy(p0_hbm.at[pl.ds(off, CT)], i0s[(c + 1) % 2])
        pltpu.sync_copy(p1_hbm.at[pl.ds(off, CT)], i1s[(c + 1) % 2])
        loads[c + 1] = pltpu.async_copy(
            hs_hbm.at[pl.ds(off, CT)], bufs[(c + 1) % 2], lsems[(c + 1) % 2])
      loads[c].wait()
      scats[c] = (
          pltpu.async_copy(bufs[pb], out_hbm.at[i0s[pb]], ssems[pb]),
          pltpu.async_copy(bufs[pb], out_hbm.at[i1s[pb]], ssems[pb]),
      )
    for c in (nchunks - 2, nchunks - 1):
      scats[c][0].wait()
      scats[c][1].wait()

  return scatter_kernel(hs, p0, p1)


def _gmm(x_sorted, w1, w2, block_expert):
  def body(be_ref, x_ref, w1_ref, w2_ref, out_ref):
    x = x_ref[...].astype(jnp.bfloat16)
    w1e = w1_ref[0].astype(jnp.bfloat16)
    h = lax.dot_general(x, w1e, (((1,), (1,)), ((), ())),
                        preferred_element_type=jnp.float32)
    gate = h[:, :FF]
    up = h[:, FF:]
    act = (gate * jax.nn.sigmoid(gate)) * up
    w2e = w2_ref[0].astype(jnp.bfloat16)
    o = lax.dot_general(act.astype(jnp.bfloat16), w2e,
                        (((1,), (1,)), ((), ())),
                        preferred_element_type=jnp.float32)
    out_ref[...] = o

  grid_spec = pltpu.PrefetchScalarGridSpec(
      num_scalar_prefetch=1,
      grid=(NB,),
      in_specs=[
          pl.BlockSpec((B, K), lambda b, be: (b, 0)),
          pl.BlockSpec((1, N, K), lambda b, be: (be[b], 0, 0)),
          pl.BlockSpec((1, K, FF), lambda b, be: (be[b], 0, 0)),
      ],
      out_specs=pl.BlockSpec((B, K), lambda b, be: (b, 0)),
  )
  return pl.pallas_call(
      body,
      grid_spec=grid_spec,
      out_shape=jax.ShapeDtypeStruct((NBB, K), jnp.float32),
  )(block_expert, x_sorted, w1, w2)


def _sc_combine(out_sorted, p0, p1, tw0, tw1):
  nchunks = TPW // CT

  @functools.partial(
      pl.kernel,
      out_type=jax.ShapeDtypeStruct((M, K), jnp.float32),
      mesh=plsc.VectorSubcoreMesh(**_MESH),
      scratch_types=[
          pltpu.VMEM((TPW,), jnp.int32),
          pltpu.VMEM((TPW,), jnp.int32),
          pltpu.VMEM((TPW,), jnp.float32),
          pltpu.VMEM((TPW,), jnp.float32),
          pltpu.VMEM((CT, K), jnp.float32),
          pltpu.VMEM((CT, K), jnp.float32),
          pltpu.VMEM((CT, K), jnp.float32),
          pltpu.VMEM((CT, K), jnp.float32),
          pltpu.SemaphoreType.DMA,
          pltpu.SemaphoreType.DMA,
          pltpu.SemaphoreType.DMA,
          pltpu.SemaphoreType.DMA,
      ],
  )
  def combine_kernel(o_hbm, p0_hbm, p1_hbm, tw0_hbm, tw1_hbm, y_hbm,
                     i0, i1, wv0, wv1,
                     bufa0, bufb0, bufa1, bufb1, sg0, sg1, so0, so1):
    wid = lax.axis_index("s") * NC + lax.axis_index("c")
    base = wid * TPW
    pltpu.sync_copy(p0_hbm.at[pl.ds(base, TPW)], i0)
    pltpu.sync_copy(p1_hbm.at[pl.ds(base, TPW)], i1)
    pltpu.sync_copy(tw0_hbm.at[pl.ds(base, TPW)], wv0)
    pltpu.sync_copy(tw1_hbm.at[pl.ds(base, TPW)], wv1)
    bufas = (bufa0, bufa1)
    bufbs = (bufb0, bufb1)
    gsems = (sg0, sg1)
    osems = (so0, so1)

    def fire(c):
      sl = pl.ds(c * CT, CT)
      pb = c % 2
      cpa = pltpu.async_copy(o_hbm.at[i0.at[sl]], bufas[pb], gsems[pb])
      cpb = pltpu.async_copy(o_hbm.at[i1.at[sl]], bufbs[pb], gsems[pb])
      return (cpa, cpb)

    gathers = [None] * nchunks
    outs = [None] * nchunks
    gathers[0] = fire(0)
    for c in range(nchunks):
      if c + 1 < nchunks:
        if c >= 1:
          outs[c - 1].wait()
        gathers[c + 1] = fire(c + 1)
      gathers[c][0].wait()
      gathers[c][1].wait()
      pb = c % 2
      bufa, bufb = bufas[pb], bufbs[pb]
      v0 = wv0[pl.ds(c * CT, CT)]
      v1 = wv1[pl.ds(c * CT, CT)]
      for r in range(CT):
        s0 = v0[r]
        s1 = v1[r]

        @plsc.parallel_loop(0, K, step=16, unroll=4)
        def _(jv, _r=r, _a=bufa, _b=bufb, _s0=s0, _s1=s1):
          sl = pl.ds(jv, 16)
          _a[_r, sl] = _a[_r, sl] * _s0 + _b[_r, sl] * _s1
      outs[c] = pltpu.async_copy(
          bufa, y_hbm.at[pl.ds(base + c * CT, CT)], osems[pb])
    outs[nchunks - 2].wait()
    outs[nchunks - 1].wait()

  return combine_kernel(out_sorted, p0, p1, tw0, tw1)


def kernel(hidden_states, w1, w2, topk_weights, topk_ids):
  flat_ids = topk_ids.reshape(-1)
  ids_e = jnp.arange(E, dtype=jnp.int32)

  onehot = (flat_ids[:, None] == ids_e[None, :]).astype(jnp.int32)
  csum = jnp.cumsum(onehot, axis=0)
  counts = csum[-1]
  padded = ((counts + B - 1) // B) * B
  pad_offsets = (jnp.cumsum(padded) - padded).astype(jnp.int32)
  rank = jnp.sum(onehot * (csum - 1), axis=1)
  dest = jnp.sum(onehot * pad_offsets[None, :], axis=1) + rank

  block_expert = (jnp.sum(
      (jnp.arange(NB, dtype=jnp.int32)[:, None] * B >= pad_offsets[None, :]
       ).astype(jnp.int32), axis=1) - 1).astype(jnp.int32)
  pos_flat = dest.reshape(M, TOPK)
  p0 = pos_flat[:, 0]
  p1 = pos_flat[:, 1]
  tw0 = topk_weights[:, 0]
  tw1 = topk_weights[:, 1]

  x_sorted = _sc_scatter(hidden_states, p0, p1)
  out_sorted = _gmm(x_sorted, w1, w2, block_expert)
  return _sc_combine(out_sorted, p0, p1, tw0, tw1)


# speedup vs baseline: 1.3120x; 1.3120x over previous
import functools

import jax
import jax.numpy as jnp
from jax import lax
from jax.experimental import pallas as pl
from jax.experimental.pallas import tpu as pltpu
from jax.experimental.pallas import tpu_sc as plsc

E = 8
TOPK = 2
M = 2048
K = 1024
FF = 1024
N = 2 * FF

B = 512
NB = (M * TOPK) // B + E
NBB = NB * B

NC, NS = 2, 16
NW = NC * NS

CT = 16
TPW = M // NW

_MESH = dict(core_axis_name="c", subcore_axis_name="s")


def _sc_scatter(hs, p0, p1):
  nchunks = TPW // CT

  @functools.partial(
      pl.kernel,
      out_type=jax.ShapeDtypeStruct((NBB, K), jnp.float32),
      mesh=plsc.VectorSubcoreMesh(**_MESH),
      scratch_types=[
          pltpu.VMEM((CT,), jnp.int32),
          pltpu.VMEM((CT,), jnp.int32),
          pltpu.VMEM((CT,), jnp.int32),
          pltpu.VMEM((CT,), jnp.int32),
          pltpu.VMEM((CT, K), jnp.float32),
          pltpu.VMEM((CT, K), jnp.float32),
          pltpu.SemaphoreType.DMA,
          pltpu.SemaphoreType.DMA,
          pltpu.SemaphoreType.DMA,
          pltpu.SemaphoreType.DMA,
      ],
  )
  def scatter_kernel(hs_hbm, p0_hbm, p1_hbm, out_hbm, i0a, i0b, i1a, i1b,
                     bufa, bufb, sla, slb, ssa, ssb):
    wid = lax.axis_index("s") * NC + lax.axis_index("c")
    base = wid * TPW
    bufs = (bufa, bufb)
    i0s = (i0a, i0b)
    i1s = (i1a, i1b)
    lsems = (sla, slb)
    ssems = (ssa, ssb)
    loads = [None] * nchunks
    scats = [None] * nchunks
    pltpu.sync_copy(p0_hbm.at[pl.ds(base, CT)], i0s[0])
    pltpu.sync_copy(p1_hbm.at[pl.ds(base, CT)], i1s[0])
    loads[0] = pltpu.async_copy(hs_hbm.at[pl.ds(base, CT)], bufs[0], lsems[0])
    for c in range(nchunks):
      pb = c % 2
      if c + 1 < nchunks:
        if c >= 1:
          scats[c - 1][0].wait()
          scats[c - 1][1].wait()
        off = base + (c + 1) * CT
        pltpu.sync_copy(p0_hbm.at[pl.ds(off, CT)], i0s[(c + 1) % 2])
        pltpu.sync_copy(p1_hbm.at[pl.ds(off, CT)], i1s[(c + 1) % 2])
        loads[c + 1] = pltpu.async_copy(
            hs_hbm.at[pl.ds(off, CT)], bufs[(c + 1) % 2], lsems[(c + 1) % 2])
      loads[c].wait()
      scats[c] = (
          pltpu.async_copy(bufs[pb], out_hbm.at[i0s[pb]], ssems[pb]),
          pltpu.async_copy(bufs[pb], out_hbm.at[i1s[pb]], ssems[pb]),
      )
    for c in (nchunks - 2, nchunks - 1):
      scats[c][0].wait()
      scats[c][1].wait()

  return scatter_kernel(hs, p0, p1)


def _gmm(x_sorted, w1, w2, block_expert):
  def body(be_ref, x_ref, w1_ref, w2_ref, out_ref):
    x = x_ref[...].astype(jnp.bfloat16)
    w1e = w1_ref[0].astype(jnp.bfloat16)
    h = lax.dot_general(x, w1e, (((1,), (1,)), ((), ())),
                        preferred_element_type=jnp.float32)
    gate = h[:, :FF]
    up = h[:, FF:]
    act = (gate * jax.nn.sigmoid(gate)) * up
    w2e = w2_ref[0].astype(jnp.bfloat16)
    o = lax.dot_general(act.astype(jnp.bfloat16), w2e,
                        (((1,), (1,)), ((), ())),
                        preferred_element_type=jnp.float32)
    out_ref[...] = o

  grid_spec = pltpu.PrefetchScalarGridSpec(
      num_scalar_prefetch=1,
      grid=(NB,),
      in_specs=[
          pl.BlockSpec((B, K), lambda b, be: (b, 0)),
          pl.BlockSpec((1, N, K), lambda b, be: (be[b], 0, 0)),
          pl.BlockSpec((1, K, FF), lambda b, be: (be[b], 0, 0)),
      ],
      out_specs=pl.BlockSpec((B, K), lambda b, be: (b, 0)),
  )
  return pl.pallas_call(
      body,
      grid_spec=grid_spec,
      out_shape=jax.ShapeDtypeStruct((NBB, K), jnp.float32),
  )(block_expert, x_sorted, w1, w2)


def _sc_combine(out_sorted, p0, p1, tw0, tw1):
  nchunks = TPW // CT

  @functools.partial(
      pl.kernel,
      out_type=jax.ShapeDtypeStruct((M, K), jnp.float32),
      mesh=plsc.VectorSubcoreMesh(**_MESH),
      scratch_types=[
          pltpu.VMEM((TPW,), jnp.int32),
          pltpu.VMEM((TPW,), jnp.int32),
          pltpu.VMEM((TPW,), jnp.float32),
          pltpu.VMEM((TPW,), jnp.float32),
          pltpu.VMEM((CT, K), jnp.float32),
          pltpu.VMEM((CT, K), jnp.float32),
          pltpu.VMEM((CT, K), jnp.float32),
          pltpu.VMEM((CT, K), jnp.float32),
          pltpu.SemaphoreType.DMA,
          pltpu.SemaphoreType.DMA,
          pltpu.SemaphoreType.DMA,
          pltpu.SemaphoreType.DMA,
      ],
  )
  def combine_kernel(o_hbm, p0_hbm, p1_hbm, tw0_hbm, tw1_hbm, y_hbm,
                     i0, i1, wv0, wv1,
                     bufa0, bufb0, bufa1, bufb1, sg0, sg1, so0, so1):
    wid = lax.axis_index("s") * NC + lax.axis_index("c")
    base = wid * TPW
    pltpu.sync_copy(p0_hbm.at[pl.ds(base, TPW)], i0)
    pltpu.sync_copy(p1_hbm.at[pl.ds(base, TPW)], i1)
    pltpu.sync_copy(tw0_hbm.at[pl.ds(base, TPW)], wv0)
    pltpu.sync_copy(tw1_hbm.at[pl.ds(base, TPW)], wv1)
    bufas = (bufa0, bufa1)
    bufbs = (bufb0, bufb1)
    gsems = (sg0, sg1)
    osems = (so0, so1)

    def fire(c):
      sl = pl.ds(c * CT, CT)
      pb = c % 2
      cpa = pltpu.async_copy(o_hbm.at[i0.at[sl]], bufas[pb], gsems[pb])
      cpb = pltpu.async_copy(o_hbm.at[i1.at[sl]], bufbs[pb], gsems[pb])
      return (cpa, cpb)

    gathers = [None] * nchunks
    outs = [None] * nchunks
    gathers[0] = fire(0)
    for c in range(nchunks):
      if c + 1 < nchunks:
        if c >= 1:
          outs[c - 1].wait()
        gathers[c + 1] = fire(c + 1)
      gathers[c][0].wait()
      gathers[c][1].wait()
      pb = c % 2
      bufa, bufb = bufas[pb], bufbs[pb]
      v0 = wv0[pl.ds(c * CT, CT)]
      v1 = wv1[pl.ds(c * CT, CT)]
      for r in range(CT):
        s0 = v0[r]
        s1 = v1[r]

        @plsc.parallel_loop(0, K, step=16, unroll=4)
        def _(jv, _r=r, _a=bufa, _b=bufb, _s0=s0, _s1=s1):
          sl = pl.ds(jv, 16)
          _a[_r, sl] = _a[_r, sl] * _s0 + _b[_r, sl] * _s1
      outs[c] = pltpu.async_copy(
          bufa, y_hbm.at[pl.ds(base + c * CT, CT)], osems[pb])
    outs[nchunks - 2].wait()
    outs[nchunks - 1].wait()

  return combine_kernel(out_sorted, p0, p1, tw0, tw1)


def kernel(hidden_states, w1, w2, topk_weights, topk_ids):
  flat_ids = topk_ids.reshape(-1)
  ids_e = jnp.arange(E, dtype=jnp.int32)

  onehot = (flat_ids[:, None] == ids_e[None, :]).astype(jnp.int32)
  csum = jnp.cumsum(onehot, axis=0)
  counts = csum[-1]
  padded = ((counts + B - 1) // B) * B
  pad_offsets = (jnp.cumsum(padded) - padded).astype(jnp.int32)
  rank = jnp.sum(onehot * (csum - 1), axis=1)
  dest = jnp.sum(onehot * pad_offsets[None, :], axis=1) + rank

  block_expert = (jnp.sum(
      (jnp.arange(NB, dtype=jnp.int32)[:, None] * B >= pad_offsets[None, :]
       ).astype(jnp.int32), axis=1) - 1).astype(jnp.int32)
  pos_flat = dest.reshape(M, TOPK)
  p0 = pos_flat[:, 0]
  p1 = pos_flat[:, 1]
  tw0 = topk_weights[:, 0]
  tw1 = topk_weights[:, 1]

  x_sorted = _sc_scatter(hidden_states, p0, p1)
  out_sorted = _gmm(x_sorted, w1, w2, block_expert)
  return _sc_combine(out_sorted, p0, p1, tw0, tw1)


# speedup vs baseline: 1.4399x; 1.0975x over previous
import functools

import jax
import jax.numpy as jnp
from jax import lax
from jax.experimental import pallas as pl
from jax.experimental.pallas import tpu as pltpu
from jax.experimental.pallas import tpu_sc as plsc

E = 8
TOPK = 2
M = 2048
K = 1024
FF = 1024
N = 2 * FF

B = 256
NB = (M * TOPK) // B + E
NBB = NB * B

NC, NS = 2, 16
NW = NC * NS

CT = 16
TPW = M // NW

_MESH = dict(core_axis_name="c", subcore_axis_name="s")


def _sc_scatter(hs, p0, p1):
  nchunks = TPW // CT

  @functools.partial(
      pl.kernel,
      out_type=jax.ShapeDtypeStruct((NBB, K), jnp.float32),
      mesh=plsc.VectorSubcoreMesh(**_MESH),
      scratch_types=[
          pltpu.VMEM((CT,), jnp.int32),
          pltpu.VMEM((CT,), jnp.int32),
          pltpu.VMEM((CT,), jnp.int32),
          pltpu.VMEM((CT,), jnp.int32),
          pltpu.VMEM((CT, K), jnp.float32),
          pltpu.VMEM((CT, K), jnp.float32),
          pltpu.SemaphoreType.DMA,
          pltpu.SemaphoreType.DMA,
          pltpu.SemaphoreType.DMA,
          pltpu.SemaphoreType.DMA,
      ],
  )
  def scatter_kernel(hs_hbm, p0_hbm, p1_hbm, out_hbm, i0a, i0b, i1a, i1b,
                     bufa, bufb, sla, slb, ssa, ssb):
    wid = lax.axis_index("s") * NC + lax.axis_index("c")
    base = wid * TPW
    bufs = (bufa, bufb)
    i0s = (i0a, i0b)
    i1s = (i1a, i1b)
    lsems = (sla, slb)
    ssems = (ssa, ssb)
    loads = [None] * nchunks
    scats = [None] * nchunks
    pltpu.sync_copy(p0_hbm.at[pl.ds(base, CT)], i0s[0])
    pltpu.sync_copy(p1_hbm.at[pl.ds(base, CT)], i1s[0])
    loads[0] = pltpu.async_copy(hs_hbm.at[pl.ds(base, CT)], bufs[0], lsems[0])
    for c in range(nchunks):
      pb = c % 2
      if c + 1 < nchunks:
        if c >= 1:
          scats[c - 1][0].wait()
          scats[c - 1][1].wait()
        off = base + (c + 1) * CT
        pltpu.sync_copy(p0_hbm.at[pl.ds(off, CT)], i0s[(c + 1) % 2])
        pltpu.sync_copy(p1_hbm.at[pl.ds(off, CT)], i1s[(c + 1) % 2])
        loads[c + 1] = pltpu.async_copy(
            hs_hbm.at[pl.ds(off, CT)], bufs[(c + 1) % 2], lsems[(c + 1) % 2])
      loads[c].wait()
      scats[c] = (
          pltpu.async_copy(bufs[pb], out_hbm.at[i0s[pb]], ssems[pb]),
          pltpu.async_copy(bufs[pb], out_hbm.at[i1s[pb]], ssems[pb]),
      )
    for c in (nchunks - 2, nchunks - 1):
      scats[c][0].wait()
      scats[c][1].wait()

  return scatter_kernel(hs, p0, p1)


def _gmm(x_sorted, w1, w2, block_expert):
  be = block_expert
  fs = jnp.concatenate(
      [jnp.ones((1,), jnp.int32), (be[1:] != be[:-1]).astype(jnp.int32)])
  rix = jnp.cumsum(fs)
  slot_arr = ((rix - 1) % 2).astype(jnp.int32)
  bidx = jnp.arange(NB, dtype=jnp.int32)
  starts = jnp.where(fs == 1, bidx, 2 * NB)
  next_start = jnp.flip(jax.lax.cummin(jnp.flip(
      jnp.concatenate([starts[1:], jnp.full((1,), 2 * NB, jnp.int32)]))))
  nxt = be[jnp.clip(next_start, 0, NB - 1)]

  def body(fs_ref, slot_ref, nxt_ref, be_ref, x_ref, w1_hbm, w2_hbm, out_ref,
           w1s, w2s, sem1, sem2):
    b = pl.program_id(0)
    cur = slot_ref[b]
    other = 1 - cur

    def w_copy(e, sl):
      return (pltpu.make_async_copy(w1_hbm.at[e], w1s.at[sl], sem1.at[sl]),
              pltpu.make_async_copy(w2_hbm.at[e], w2s.at[sl], sem2.at[sl]))

    @pl.when(b == 0)
    def _():
      for cp in w_copy(be_ref[0], cur):
        cp.start()

    @pl.when(fs_ref[b] == 1)
    def _():
      for cp in w_copy(nxt_ref[b], other):
        cp.start()
      for cp in w_copy(be_ref[b], cur):
        cp.wait()

    x = x_ref[...].astype(jnp.bfloat16)
    w1e = w1s[cur].astype(jnp.bfloat16)
    h = lax.dot_general(x, w1e, (((1,), (1,)), ((), ())),
                        preferred_element_type=jnp.float32)
    gate = h[:, :FF]
    up = h[:, FF:]
    act = (gate * jax.nn.sigmoid(gate)) * up
    w2e = w2s[cur].astype(jnp.bfloat16)
    o = lax.dot_general(act.astype(jnp.bfloat16), w2e,
                        (((1,), (1,)), ((), ())),
                        preferred_element_type=jnp.float32)
    out_ref[...] = o

    @pl.when(b == NB - 1)
    def _():
      for cp in w_copy(nxt_ref[b], other):
        cp.wait()

  grid_spec = pltpu.PrefetchScalarGridSpec(
      num_scalar_prefetch=4,
      grid=(NB,),
      in_specs=[
          pl.BlockSpec((B, K), lambda b, *_: (b, 0)),
          pl.BlockSpec(memory_space=pl.ANY),
          pl.BlockSpec(memory_space=pl.ANY),
      ],
      out_specs=pl.BlockSpec((B, K), lambda b, *_: (b, 0)),
      scratch_shapes=[
          pltpu.VMEM((2, N, K), jnp.float32),
          pltpu.VMEM((2, K, FF), jnp.float32),
          pltpu.SemaphoreType.DMA((2,)),
          pltpu.SemaphoreType.DMA((2,)),
      ],
  )
  return pl.pallas_call(
      body,
      grid_spec=grid_spec,
      out_shape=jax.ShapeDtypeStruct((NBB, K), jnp.float32),
  )(fs, slot_arr, nxt, be, x_sorted, w1, w2)


def _sc_combine(out_sorted, p0, p1, tw0, tw1):
  nchunks = TPW // CT

  @functools.partial(
      pl.kernel,
      out_type=jax.ShapeDtypeStruct((M, K), jnp.float32),
      mesh=plsc.VectorSubcoreMesh(**_MESH),
      scratch_types=[
          pltpu.VMEM((TPW,), jnp.int32),
          pltpu.VMEM((TPW,), jnp.int32),
          pltpu.VMEM((TPW,), jnp.float32),
          pltpu.VMEM((TPW,), jnp.float32),
          pltpu.VMEM((CT, K), jnp.float32),
          pltpu.VMEM((CT, K), jnp.float32),
          pltpu.VMEM((CT, K), jnp.float32),
          pltpu.VMEM((CT, K), jnp.float32),
          pltpu.SemaphoreType.DMA,
          pltpu.SemaphoreType.DMA,
          pltpu.SemaphoreType.DMA,
          pltpu.SemaphoreType.DMA,
      ],
  )
  def combine_kernel(o_hbm, p0_hbm, p1_hbm, tw0_hbm, tw1_hbm, y_hbm,
                     i0, i1, wv0, wv1,
                     bufa0, bufb0, bufa1, bufb1, sg0, sg1, so0, so1):
    wid = lax.axis_index("s") * NC + lax.axis_index("c")
    base = wid * TPW
    pltpu.sync_copy(p0_hbm.at[pl.ds(base, TPW)], i0)
    pltpu.sync_copy(p1_hbm.at[pl.ds(base, TPW)], i1)
    pltpu.sync_copy(tw0_hbm.at[pl.ds(base, TPW)], wv0)
    pltpu.sync_copy(tw1_hbm.at[pl.ds(base, TPW)], wv1)
    bufas = (bufa0, bufa1)
    bufbs = (bufb0, bufb1)
    gsems = (sg0, sg1)
    osems = (so0, so1)

    def fire(c):
      sl = pl.ds(c * CT, CT)
      pb = c % 2
      cpa = pltpu.async_copy(o_hbm.at[i0.at[sl]], bufas[pb], gsems[pb])
      cpb = pltpu.async_copy(o_hbm.at[i1.at[sl]], bufbs[pb], gsems[pb])
      return (cpa, cpb)

    gathers = [None] * nchunks
    outs = [None] * nchunks
    gathers[0] = fire(0)
    for c in range(nchunks):
      if c + 1 < nchunks:
        if c >= 1:
          outs[c - 1].wait()
        gathers[c + 1] = fire(c + 1)
      gathers[c][0].wait()
      gathers[c][1].wait()
      pb = c % 2
      bufa, bufb = bufas[pb], bufbs[pb]
      v0 = wv0[pl.ds(c * CT, CT)]
      v1 = wv1[pl.ds(c * CT, CT)]
      for r in range(CT):
        s0 = v0[r]
        s1 = v1[r]

        @plsc.parallel_loop(0, K, step=16, unroll=4)
        def _(jv, _r=r, _a=bufa, _b=bufb, _s0=s0, _s1=s1):
          sl = pl.ds(jv, 16)
          _a[_r, sl] = _a[_r, sl] * _s0 + _b[_r, sl] * _s1
      outs[c] = pltpu.async_copy(
          bufa, y_hbm.at[pl.ds(base + c * CT, CT)], osems[pb])
    outs[nchunks - 2].wait()
    outs[nchunks - 1].wait()

  return combine_kernel(out_sorted, p0, p1, tw0, tw1)


def kernel(hidden_states, w1, w2, topk_weights, topk_ids):
  flat_ids = topk_ids.reshape(-1)
  ids_e = jnp.arange(E, dtype=jnp.int32)

  onehot = (flat_ids[:, None] == ids_e[None, :]).astype(jnp.int32)
  csum = jnp.cumsum(onehot, axis=0)
  counts = csum[-1]
  padded = ((counts + B - 1) // B) * B
  pad_offsets = (jnp.cumsum(padded) - padded).astype(jnp.int32)
  rank = jnp.sum(onehot * (csum - 1), axis=1)
  dest = jnp.sum(onehot * pad_offsets[None, :], axis=1) + rank

  block_expert = (jnp.sum(
      (jnp.arange(NB, dtype=jnp.int32)[:, None] * B >= pad_offsets[None, :]
       ).astype(jnp.int32), axis=1) - 1).astype(jnp.int32)
  pos_flat = dest.reshape(M, TOPK)
  p0 = pos_flat[:, 0]
  p1 = pos_flat[:, 1]
  tw0 = topk_weights[:, 0]
  tw1 = topk_weights[:, 1]

  x_sorted = _sc_scatter(hidden_states, p0, p1)
  out_sorted = _gmm(x_sorted, w1, w2, block_expert)
  return _sc_combine(out_sorted, p0, p1, tw0, tw1)


# speedup vs baseline: 1.4569x; 1.0118x over previous
import functools

import jax
import jax.numpy as jnp
from jax import lax
from jax.experimental import pallas as pl
from jax.experimental.pallas import tpu as pltpu
from jax.experimental.pallas import tpu_sc as plsc

E = 8
TOPK = 2
M = 2048
K = 1024
FF = 1024
N = 2 * FF

B = 256
NB = (M * TOPK) // B + E
NBB = NB * B

NC, NS = 2, 16
NW = NC * NS

CT = 16
TPW = M // NW

_MESH = dict(core_axis_name="c", subcore_axis_name="s")


def _sc_scatter(hs, p0, p1):
  nchunks = TPW // CT

  @functools.partial(
      pl.kernel,
      out_type=jax.ShapeDtypeStruct((NBB, K), jnp.float32),
      mesh=plsc.VectorSubcoreMesh(**_MESH),
      scratch_types=(
          [pltpu.VMEM((CT,), jnp.int32)] * 6 +
          [pltpu.VMEM((CT, K), jnp.float32)] * 3 +
          [pltpu.SemaphoreType.DMA] * 6
      ),
  )
  def scatter_kernel(hs_hbm, p0_hbm, p1_hbm, out_hbm,
                     i0a, i0b, i0c, i1a, i1b, i1c, bufa, bufb, bufc,
                     sla, slb, slc, ssa, ssb, ssc):
    wid = lax.axis_index("s") * NC + lax.axis_index("c")
    base = wid * TPW
    bufs = (bufa, bufb, bufc)
    i0s = (i0a, i0b, i0c)
    i1s = (i1a, i1b, i1c)
    lsems = (sla, slb, slc)
    ssems = (ssa, ssb, ssc)
    loads = [None] * nchunks
    scats = [None] * nchunks

    def fire(c):
      off = base + c * CT
      pb = c % 3
      pltpu.sync_copy(p0_hbm.at[pl.ds(off, CT)], i0s[pb])
      pltpu.sync_copy(p1_hbm.at[pl.ds(off, CT)], i1s[pb])
      loads[c] = pltpu.async_copy(hs_hbm.at[pl.ds(off, CT)], bufs[pb],
                                  lsems[pb])

    fire(0)
    fire(1)
    for c in range(nchunks):
      pb = c % 3
      if c + 2 < nchunks:
        if c >= 1:
          scats[c - 1][0].wait()
          scats[c - 1][1].wait()
        fire(c + 2)
      loads[c].wait()
      scats[c] = (
          pltpu.async_copy(bufs[pb], out_hbm.at[i0s[pb]], ssems[pb]),
          pltpu.async_copy(bufs[pb], out_hbm.at[i1s[pb]], ssems[pb]),
      )
    for c in range(max(0, nchunks - 3), nchunks):
      scats[c][0].wait()
      scats[c][1].wait()

  return scatter_kernel(hs, p0, p1)


def _gmm(x_sorted, w1, w2, block_expert):
  be = block_expert
  fs = jnp.concatenate(
      [jnp.ones((1,), jnp.int32), (be[1:] != be[:-1]).astype(jnp.int32)])
  rix = jnp.cumsum(fs)
  slot_arr = ((rix - 1) % 2).astype(jnp.int32)
  bidx = jnp.arange(NB, dtype=jnp.int32)
  starts = jnp.where(fs == 1, bidx, 2 * NB)
  next_start = jnp.flip(jax.lax.cummin(jnp.flip(
      jnp.concatenate([starts[1:], jnp.full((1,), 2 * NB, jnp.int32)]))))
  nxt = be[jnp.clip(next_start, 0, NB - 1)]

  def body(fs_ref, slot_ref, nxt_ref, be_ref, x_ref, w1_hbm, w2_hbm, out_ref,
           w1s, w2s, sem1, sem2):
    b = pl.program_id(0)
    cur = slot_ref[b]
    other = 1 - cur

    def w_copy(e, sl):
      return (pltpu.make_async_copy(w1_hbm.at[e], w1s.at[sl], sem1.at[sl]),
              pltpu.make_async_copy(w2_hbm.at[e], w2s.at[sl], sem2.at[sl]))

    @pl.when(b == 0)
    def _():
      for cp in w_copy(be_ref[0], cur):
        cp.start()

    @pl.when(fs_ref[b] == 1)
    def _():
      for cp in w_copy(nxt_ref[b], other):
        cp.start()
      for cp in w_copy(be_ref[b], cur):
        cp.wait()

    x = x_ref[...].astype(jnp.bfloat16)
    w1e = w1s[cur].astype(jnp.bfloat16)
    h = lax.dot_general(x, w1e, (((1,), (1,)), ((), ())),
                        preferred_element_type=jnp.float32)
    gate = h[:, :FF]
    up = h[:, FF:]
    act = (gate * jax.nn.sigmoid(gate)) * up
    w2e = w2s[cur].astype(jnp.bfloat16)
    o = lax.dot_general(act.astype(jnp.bfloat16), w2e,
                        (((1,), (1,)), ((), ())),
                        preferred_element_type=jnp.float32)
    out_ref[...] = o

    @pl.when(b == NB - 1)
    def _():
      for cp in w_copy(nxt_ref[b], other):
        cp.wait()

  grid_spec = pltpu.PrefetchScalarGridSpec(
      num_scalar_prefetch=4,
      grid=(NB,),
      in_specs=[
          pl.BlockSpec((B, K), lambda b, *_: (b, 0)),
          pl.BlockSpec(memory_space=pl.ANY),
          pl.BlockSpec(memory_space=pl.ANY),
      ],
      out_specs=pl.BlockSpec((B, K), lambda b, *_: (b, 0)),
      scratch_shapes=[
          pltpu.VMEM((2, N, K), jnp.float32),
          pltpu.VMEM((2, K, FF), jnp.float32),
          pltpu.SemaphoreType.DMA((2,)),
          pltpu.SemaphoreType.DMA((2,)),
      ],
  )
  return pl.pallas_call(
      body,
      grid_spec=grid_spec,
      out_shape=jax.ShapeDtypeStruct((NBB, K), jnp.float32),
  )(fs, slot_arr, nxt, be, x_sorted, w1, w2)


def _sc_combine(out_sorted, p0, p1, tw0, tw1):
  nchunks = TPW // CT

  @functools.partial(
      pl.kernel,
      out_type=jax.ShapeDtypeStruct((M, K), jnp.float32),
      mesh=plsc.VectorSubcoreMesh(**_MESH),
      scratch_types=(
          [pltpu.VMEM((TPW,), jnp.int32)] * 2 +
          [pltpu.VMEM((TPW,), jnp.float32)] * 2 +
          [pltpu.VMEM((CT, K), jnp.float32)] * 6 +
          [pltpu.SemaphoreType.DMA] * 6
      ),
  )
  def combine_kernel(o_hbm, p0_hbm, p1_hbm, tw0_hbm, tw1_hbm, y_hbm,
                     i0, i1, wv0, wv1,
                     bufa0, bufa1, bufa2, bufb0, bufb1, bufb2,
                     sg0, sg1, sg2, so0, so1, so2):
    wid = lax.axis_index("s") * NC + lax.axis_index("c")
    base = wid * TPW
    pltpu.sync_copy(p0_hbm.at[pl.ds(base, TPW)], i0)
    pltpu.sync_copy(p1_hbm.at[pl.ds(base, TPW)], i1)
    pltpu.sync_copy(tw0_hbm.at[pl.ds(base, TPW)], wv0)
    pltpu.sync_copy(tw1_hbm.at[pl.ds(base, TPW)], wv1)
    bufas = (bufa0, bufa1, bufa2)
    bufbs = (bufb0, bufb1, bufb2)
    gsems = (sg0, sg1, sg2)
    osems = (so0, so1, so2)

    def fire(c):
      sl = pl.ds(c * CT, CT)
      pb = c % 3
      cpa = pltpu.async_copy(o_hbm.at[i0.at[sl]], bufas[pb], gsems[pb])
      cpb = pltpu.async_copy(o_hbm.at[i1.at[sl]], bufbs[pb], gsems[pb])
      return (cpa, cpb)

    gathers = [None] * nchunks
    outs = [None] * nchunks
    gathers[0] = fire(0)
    gathers[1] = fire(1)
    for c in range(nchunks):
      if c + 2 < nchunks:
        if c >= 1:
          outs[c - 1].wait()
        gathers[c + 2] = fire(c + 2)
      gathers[c][0].wait()
      gathers[c][1].wait()
      pb = c % 3
      bufa, bufb = bufas[pb], bufbs[pb]
      v0 = wv0[pl.ds(c * CT, CT)]
      v1 = wv1[pl.ds(c * CT, CT)]
      for r in range(CT):
        s0 = v0[r]
        s1 = v1[r]

        @plsc.parallel_loop(0, K, step=16, unroll=4)
        def _(jv, _r=r, _a=bufa, _b=bufb, _s0=s0, _s1=s1):
          sl = pl.ds(jv, 16)
          _a[_r, sl] = _a[_r, sl] * _s0 + _b[_r, sl] * _s1
      outs[c] = pltpu.async_copy(
          bufa, y_hbm.at[pl.ds(base + c * CT, CT)], osems[pb])
    for c in range(max(0, nchunks - 3), nchunks):
      outs[c].wait()

  return combine_kernel(out_sorted, p0, p1, tw0, tw1)


def kernel(hidden_states, w1, w2, topk_weights, topk_ids):
  flat_ids = topk_ids.reshape(-1)
  ids_e = jnp.arange(E, dtype=jnp.int32)

  onehot = (flat_ids[:, None] == ids_e[None, :]).astype(jnp.int32)
  csum = jnp.cumsum(onehot, axis=0)
  counts = csum[-1]
  padded = ((counts + B - 1) // B) * B
  pad_offsets = (jnp.cumsum(padded) - padded).astype(jnp.int32)
  rank = jnp.sum(onehot * (csum - 1), axis=1)
  dest = jnp.sum(onehot * pad_offsets[None, :], axis=1) + rank

  block_expert = (jnp.sum(
      (jnp.arange(NB, dtype=jnp.int32)[:, None] * B >= pad_offsets[None, :]
       ).astype(jnp.int32), axis=1) - 1).astype(jnp.int32)
  pos_flat = dest.reshape(M, TOPK)
  p0 = pos_flat[:, 0]
  p1 = pos_flat[:, 1]
  tw0 = topk_weights[:, 0]
  tw1 = topk_weights[:, 1]

  x_sorted = _sc_scatter(hidden_states, p0, p1)
  out_sorted = _gmm(x_sorted, w1, w2, block_expert)
  return _sc_combine(out_sorted, p0, p1, tw0, tw1)
